# baseline (speedup 1.0000x reference)
import functools
import math

import jax
import jax.numpy as jnp
from jax import lax
from jax.experimental import pallas as pl
from jax.experimental.pallas import tpu as pltpu
from jax.experimental.pallas import tpu_sc as plsc

S = 4096
B = 4
D = 1024
LANES = 16
SCALE = 1.0 / math.sqrt(D)

NC = 2
NS = 16
NW = NC * NS
S_PER_W = S // NW
CH = 8
NSTEPS = S_PER_W // CH
NBUF = 3


def _body(x_hbm, pos_hbm, out_hbm, xbuf, pbuf,
          xsem, psem, osem):
    wid = lax.axis_index("s") * NC + lax.axis_index("c")
    base = wid * S_PER_W

    def compute(slot, half):
        nvec = D // LANES
        h = CH // 2

        @plsc.parallel_loop(half * h * nvec, (half + 1) * h * nvec, unroll=4)
        def _(j):
            i = j // nvec
            d = (j % nvec) * LANES
            pv = pbuf[slot, i, pl.ds(d, LANES)] * SCALE
            for b in range(B):
                plsc.addupdate(xbuf.at[slot, i, b, pl.ds(d, LANES)], pv)

    copies = {}

    def start_in(step):
        slot = step % NBUF
        s0 = base + step * CH
        copies[("x", step)] = pltpu.async_copy(
            x_hbm.at[pl.ds(s0, CH)], xbuf.at[slot], xsem.at[slot])
        copies[("p", step)] = pltpu.async_copy(
            pos_hbm.at[pl.ds(s0, CH)], pbuf.at[slot], psem.at[slot])

    start_in(0)
    for step in range(NSTEPS):
        slot = step % NBUF
        if step + 1 < NSTEPS:
            if step + 1 >= NBUF:
                copies[("oa", step + 1 - NBUF)].wait()
                copies[("o", step + 1 - NBUF)].wait()
            start_in(step + 1)
        copies[("x", step)].wait()
        copies[("p", step)].wait()
        compute(slot, 0)
        copies[("oa", step)] = pltpu.async_copy(
            xbuf.at[slot, pl.ds(0, CH // 2)],
            out_hbm.at[pl.ds(base + step * CH, CH // 2)],
            osem.at[slot])
        compute(slot, 1)
        copies[("o", step)] = pltpu.async_copy(
            xbuf.at[slot, pl.ds(CH // 2, CH // 2)],
            out_hbm.at[pl.ds(base + step * CH + CH // 2, CH // 2)],
            osem.at[slot])
    for step in range(NSTEPS - NBUF + 1, NSTEPS):
        copies[("oa", step - 1)].wait()
        copies[("o", step - 1)].wait()
    copies[("oa", NSTEPS - 1)].wait()
    copies[("o", NSTEPS - 1)].wait()


@jax.jit
def kernel(x, pos_embedding):
    mesh = plsc.VectorSubcoreMesh(core_axis_name="c", subcore_axis_name="s")
    run = functools.partial(
        pl.kernel,
        mesh=mesh,
        out_type=jax.ShapeDtypeStruct((S, B, D), jnp.float32),
        scratch_types=[
            pltpu.VMEM((NBUF, CH, B, D), jnp.float32),
            pltpu.VMEM((NBUF, CH, D), jnp.float32),
            pltpu.SemaphoreType.DMA((NBUF,)),
            pltpu.SemaphoreType.DMA((NBUF,)),
            pltpu.SemaphoreType.DMA((NBUF,)),
        ],
    )(_body)
    return run(x, pos_embedding)


# speedup vs baseline: 1.0284x; 1.0284x over previous
import functools
import math

import jax
import jax.numpy as jnp
from jax import lax
from jax.experimental import pallas as pl
from jax.experimental.pallas import tpu as pltpu
from jax.experimental.pallas import tpu_sc as plsc

S = 4096
B = 4
D = 1024
LANES = 16
SCALE = 1.0 / math.sqrt(D)

NC = 2
NS = 16
NW = NC * NS
S_PER_W = S // NW
CH = 8
NSTEPS = S_PER_W // CH
NBUF = 3


def _body(x_hbm, pos_hbm, out_hbm, xbuf, pbuf,
          xsem, psem, osem):
    wid = lax.axis_index("s") * NC + lax.axis_index("c")
    base = wid * S_PER_W

    def compute(slot):
        nvec = D // LANES

        @plsc.parallel_loop(0, CH * nvec, unroll=4)
        def _(j):
            i = j // nvec
            d = (j % nvec) * LANES
            pv = pbuf[slot, i, pl.ds(d, LANES)] * SCALE
            for b in range(B):
                plsc.addupdate(xbuf.at[slot, i, b, pl.ds(d, LANES)], pv)

    copies = {}

    def start_in(step):
        slot = step % NBUF
        s0 = base + step * CH
        copies[("x", step)] = pltpu.async_copy(
            x_hbm.at[pl.ds(s0, CH)], xbuf.at[slot], xsem.at[slot])
        copies[("p", step)] = pltpu.async_copy(
            pos_hbm.at[pl.ds(s0, CH)], pbuf.at[slot], psem.at[slot])

    start_in(0)
    for step in range(NSTEPS):
        slot = step % NBUF
        if step + 1 < NSTEPS:
            if step + 1 >= NBUF:
                copies[("o", step + 1 - NBUF)].wait()
            start_in(step + 1)
        copies[("x", step)].wait()
        copies[("p", step)].wait()
        compute(slot)
        copies[("o", step)] = pltpu.async_copy(
            xbuf.at[slot], out_hbm.at[pl.ds(base + step * CH, CH)],
            osem.at[slot])
    for step in range(NSTEPS - NBUF + 1, NSTEPS):
        copies[("o", step - 1)].wait()
    copies[("o", NSTEPS - 1)].wait()


@jax.jit
def kernel(x, pos_embedding):
    mesh = plsc.VectorSubcoreMesh(core_axis_name="c", subcore_axis_name="s")
    run = functools.partial(
        pl.kernel,
        mesh=mesh,
        out_type=jax.ShapeDtypeStruct((S, B, D), jnp.float32),
        scratch_types=[
            pltpu.VMEM((NBUF, CH, B, D), jnp.float32),
            pltpu.VMEM((NBUF, CH, D), jnp.float32),
            pltpu.SemaphoreType.DMA((NBUF,)),
            pltpu.SemaphoreType.DMA((NBUF,)),
            pltpu.SemaphoreType.DMA((NBUF,)),
        ],
    )(_body)
    return run(x, pos_embedding)
